# Initial kernel scaffold; baseline (speedup 1.0000x reference)
import jax
import jax.numpy as jnp
from jax.experimental import pallas as pl


def kernel(hidden_states, topk_weights, w_gate_up, w_down, gate_up_lora_a, gate_up_lora_b, down_lora_a, down_lora_b, topk_ids, token_lora_ids):
    raise NotImplementedError("write your pallas kernel here")


# speedup vs baseline: 2.3826x; 2.3826x over previous
import functools

import jax
import jax.numpy as jnp
from jax.experimental import pallas as pl
from jax.experimental.pallas import tpu as pltpu


def _moe_body(be_ref, bl_ref, rt_ref, x_ref, wgu_ref, wd_ref, gua_ref,
              gub_ref, da_ref, db_ref, rw_ref, out_ref, xs_ref, dn_ref,
              *, B, F, FC):
    i = pl.program_id(0)

    @pl.when(i == 0)
    def _init():
        out_ref[...] = jnp.zeros_like(out_ref)

    def gather_body(r, carry):
        t = rt_ref[i * B + r]
        xs_ref[r, :] = x_ref[t, :]
        return carry

    jax.lax.fori_loop(0, B, gather_body, 0)

    xs = xs_ref[...]
    gua = gua_ref[0, 0]
    gub = gub_ref[0, 0]
    da = da_ref[0, 0]
    db = db_ref[0, 0]
    wd = wd_ref[0]

    nt = (((1,), (1,)), ((), ()))
    f32 = jnp.float32

    u = jax.lax.dot_general(xs, gua, nt, preferred_element_type=f32)

    dn = jnp.zeros((B, xs.shape[1]), f32)
    v = jnp.zeros((B, gua.shape[0]), f32)
    for f0 in range(0, F, FC):
        wg = wgu_ref[0, f0:f0 + FC, :]
        wu = wgu_ref[0, F + f0:F + f0 + FC, :]
        gate = jax.lax.dot_general(xs, wg, nt, preferred_element_type=f32)
        gate += jax.lax.dot_general(u, gub[f0:f0 + FC], nt,
                                    preferred_element_type=f32)
        up = jax.lax.dot_general(xs, wu, nt, preferred_element_type=f32)
        up += jax.lax.dot_general(u, gub[F + f0:F + f0 + FC], nt,
                                  preferred_element_type=f32)
        act = gate / (1.0 + jnp.exp(-gate)) * up
        dn += jax.lax.dot_general(act, wd[:, f0:f0 + FC], nt,
                                  preferred_element_type=f32)
        v += jax.lax.dot_general(act, da[:, f0:f0 + FC], nt,
                                 preferred_element_type=f32)
    dn += jax.lax.dot_general(v, db, nt, preferred_element_type=f32)
    dn_ref[...] = dn * rw_ref[0]

    def scatter_body(r, carry):
        t = rt_ref[i * B + r]
        out_ref[t, :] = out_ref[t, :] + dn_ref[r, :]
        return carry

    jax.lax.fori_loop(0, B, scatter_body, 0)


def kernel(hidden_states, topk_weights, w_gate_up, w_down, gate_up_lora_a,
           gate_up_lora_b, down_lora_a, down_lora_b, topk_ids,
           token_lora_ids):
    T, D = hidden_states.shape
    E, two_f, _ = w_gate_up.shape
    F = two_f // 2
    L, _, R, _ = gate_up_lora_a.shape
    K = topk_ids.shape[1]
    TK = T * K
    B = 128
    FC = 512
    NG = E * L
    NB = TK // B + NG

    i32 = jnp.int32
    tw = topk_weights / jnp.sum(topk_weights, axis=-1, keepdims=True)
    flat_w = tw.reshape(-1)
    flat_e = topk_ids.reshape(-1).astype(i32)
    flat_l = jnp.broadcast_to(token_lora_ids.astype(i32)[:, None],
                              (T, K)).reshape(-1)
    g = flat_e * L + flat_l

    onehot = (g[:, None] == jnp.arange(NG, dtype=i32)[None, :]).astype(i32)
    csum = jnp.cumsum(onehot, axis=0)
    counts = csum[-1]
    rank = jnp.take_along_axis(csum, g[:, None], axis=1)[:, 0] - 1
    group_off = jnp.concatenate(
        [jnp.zeros((1,), i32), jnp.cumsum(counts)[:-1].astype(i32)])
    dest = group_off[g] + rank
    order = jnp.zeros((TK,), i32).at[dest].set(
        jnp.arange(TK, dtype=i32))

    bpg = (counts + B - 1) // B
    block_off = jnp.concatenate(
        [jnp.zeros((1,), i32), jnp.cumsum(bpg)[:-1].astype(i32)])
    bids = jnp.arange(NB, dtype=i32)
    bg = jnp.clip(jnp.searchsorted(block_off, bids, side='right').astype(i32)
                  - 1, 0, NG - 1)
    be = bg // L
    bl = bg % L

    slot_r = jnp.arange(B, dtype=i32)
    p = (bids[:, None] - block_off[bg][:, None]) * B + slot_r[None, :]
    valid = (p >= 0) & (p < counts[bg][:, None])
    pair = order[jnp.clip(group_off[bg][:, None] + p, 0, TK - 1)]
    pair = jnp.where(valid, pair, 0)
    rt = jnp.where(valid, pair // K, 0).astype(i32).reshape(-1)
    rw = jnp.where(valid, flat_w[pair], 0.0).astype(jnp.float32)
    rw = rw.reshape(NB, B, 1)

    grid_spec = pltpu.PrefetchScalarGridSpec(
        num_scalar_prefetch=3,
        grid=(NB,),
        in_specs=[
            pl.BlockSpec((T, D), lambda i, be, bl, rt: (0, 0)),
            pl.BlockSpec((1, two_f, D),
                         lambda i, be, bl, rt: (be[i], 0, 0)),
            pl.BlockSpec((1, D, F), lambda i, be, bl, rt: (be[i], 0, 0)),
            pl.BlockSpec((1, 1, R, D),
                         lambda i, be, bl, rt: (bl[i], be[i], 0, 0)),
            pl.BlockSpec((1, 1, two_f, R),
                         lambda i, be, bl, rt: (bl[i], be[i], 0, 0)),
            pl.BlockSpec((1, 1, R, F),
                         lambda i, be, bl, rt: (bl[i], be[i], 0, 0)),
            pl.BlockSpec((1, 1, D, R),
                         lambda i, be, bl, rt: (bl[i], be[i], 0, 0)),
            pl.BlockSpec((1, B, 1), lambda i, be, bl, rt: (i, 0, 0)),
        ],
        out_specs=pl.BlockSpec((T, D), lambda i, be, bl, rt: (0, 0)),
        scratch_shapes=[
            pltpu.VMEM((B, D), jnp.float32),
            pltpu.VMEM((B, D), jnp.float32),
        ],
    )
    out = pl.pallas_call(
        functools.partial(_moe_body, B=B, F=F, FC=FC),
        grid_spec=grid_spec,
        out_shape=jax.ShapeDtypeStruct((T, D), jnp.float32),
    )(be, bl, rt, hidden_states, w_gate_up, w_down, gate_up_lora_a,
      gate_up_lora_b, down_lora_a, down_lora_b, rw)
    return out
